# baseline (speedup 1.0000x reference)
import jax
import jax.numpy as jnp
from jax import lax
from jax.experimental import pallas as pl
from jax.experimental.pallas import tpu as pltpu
from jax.experimental.pallas import tpu_sc as plsc
import functools

NC = 2
NS = 16
L = 16
B = 64
S = 16
V = 100000
N = B * S
NG = B // L
NT = N // L
NQ = 8
QL = N // NQ


def _sc_loss_kernel(prob_hbm, tgt_hbm, rew_hbm, out_hbm,
                    tgt_v, idx_v, vals_v, rew_v, out_v, sem):
    c = lax.axis_index("c")
    w = lax.axis_index("s")

    @pl.when(jnp.logical_and(c == 0, w == 0))
    def _():
        pltpu.sync_copy(tgt_hbm, tgt_v)
        pltpu.sync_copy(rew_hbm, rew_v)

        iota = lax.iota(jnp.int32, L)
        for t in range(NT):
            s, g = t // NG, t % NG
            rows = (g * L + iota) * S + s
            idx_v.at[t // NQ][pl.ds((t % NQ) * L, L)] = (
                rows * V + tgt_v[pl.ds(t * L, L)])

        copies = [
            pltpu.async_copy(prob_hbm.at[idx_v.at[q]], vals_v.at[q], sem)
            for q in range(NQ)
        ]
        for cp in copies:
            cp.wait()

        total = jnp.zeros((L,), jnp.float32)
        for g in range(NG):
            acc = jnp.zeros((L,), jnp.float32)
            for s in range(S):
                t = s * NG + g
                acc = acc + vals_v.at[t // NQ][pl.ds((t % NQ) * L, L)]
            total = total + acc * rew_v[pl.ds(g * L, L)]
        out_v[...] = jnp.broadcast_to(-jnp.sum(total), (L,))
        pltpu.sync_copy(out_v, out_hbm)


@functools.partial(
    pl.kernel,
    out_type=jax.ShapeDtypeStruct((L,), jnp.float32),
    mesh=plsc.VectorSubcoreMesh(
        core_axis_name="c", subcore_axis_name="s",
        num_cores=NC, num_subcores=NS),
    compiler_params=pltpu.CompilerParams(needs_layout_passes=False),
    scratch_types=[
        pltpu.VMEM((N,), jnp.int32),
        pltpu.VMEM((NQ, QL), jnp.int32),
        pltpu.VMEM((NQ, QL), jnp.float32),
        pltpu.VMEM((B,), jnp.float32),
        pltpu.VMEM((L,), jnp.float32),
        pltpu.SemaphoreType.DMA,
    ],
)
def _sc_loss(prob_hbm, tgt_hbm, rew_hbm, out_hbm, *scratch):
    _sc_loss_kernel(prob_hbm, tgt_hbm, rew_hbm, out_hbm, *scratch)


def kernel(prob, target, reward):
    prob_flat = prob.reshape(-1)
    tgt_seq_major = target.astype(jnp.int32).T.reshape(-1)
    out = _sc_loss(prob_flat, tgt_seq_major, reward)
    return out[0]


# speedup vs baseline: 22.4901x; 22.4901x over previous
import jax
import jax.numpy as jnp
from jax import lax
from jax.experimental import pallas as pl
from jax.experimental.pallas import tpu as pltpu
from jax.experimental.pallas import tpu_sc as plsc
import functools

NC = 2
NS = 16
L = 16
B = 64
S = 16
V = 100000
N = B * S
RPW = N // (NC * NS)
W8 = 8


def _sc_gather_kernel(prob_hbm, tgt_hbm, sel_hbm, tgt_v, win_v, out_v, sem):
    w = lax.axis_index("c") * NS + lax.axis_index("s")
    base = w * RPW

    pltpu.sync_copy(tgt_hbm.at[pl.ds(base, RPW)], tgt_v)

    iota = lax.iota(jnp.int32, L)
    negi = jnp.full((L,), -1, jnp.int32)

    tks = []
    copies = []
    for k in range(RPW):
        tv = tgt_v[pl.ds((k // L) * L, L)]
        t = jnp.max(jnp.where(iota == (k % L), tv, negi))
        tks.append(t)
        tcol = pl.multiple_of(jnp.bitwise_and(t, -128), 128)
        bb = 2 * w + k // S
        sg = (k % S) // 8 * 8
        copies.append(pltpu.async_copy(
            prob_hbm.at[bb, pl.ds(sg, 8), pl.ds(tcol, 128)],
            win_v.at[k], sem))
    for cp in copies:
        cp.wait()

    ninf = jnp.full((L,), -jnp.inf, jnp.float32)
    for half in range(RPW // L):
        acc = jnp.zeros((L,), jnp.float32)
        for kk in range(L):
            k = half * L + kk
            lw = pl.multiple_of(
                jnp.bitwise_and(jnp.bitwise_and(tks[k], 127), -L), L)
            lanepos = jnp.bitwise_and(tks[k], L - 1)
            w16 = win_v.at[k, k % 8][pl.ds(lw, L)]
            v = jnp.max(jnp.where(iota == lanepos, w16, ninf))
            acc = jnp.where(iota == kk, jnp.full((L,), v), acc)
        out_v[pl.ds(half * L, L)] = acc

    pltpu.sync_copy(out_v, sel_hbm.at[pl.ds(base, RPW)])


@functools.partial(
    pl.kernel,
    out_type=jax.ShapeDtypeStruct((N,), jnp.float32),
    mesh=plsc.VectorSubcoreMesh(
        core_axis_name="c", subcore_axis_name="s",
        num_cores=NC, num_subcores=NS),
    compiler_params=pltpu.CompilerParams(
        needs_layout_passes=False, use_tc_tiling_on_sc=True),
    scratch_types=[
        pltpu.VMEM((RPW,), jnp.int32),
        pltpu.VMEM((RPW, 8, 128), jnp.float32),
        pltpu.VMEM((RPW,), jnp.float32),
        pltpu.SemaphoreType.DMA,
    ],
)
def _sc_gather(prob_hbm, tgt_hbm, sel_hbm, *scratch):
    _sc_gather_kernel(prob_hbm, tgt_hbm, sel_hbm, *scratch)


def _tc_reduce_kernel(sel_ref, rew_ref, out_ref):
    out_ref[0, 0] = -jnp.sum(sel_ref[...] * rew_ref[...])


_tc_reduce = pl.pallas_call(
    _tc_reduce_kernel,
    out_shape=jax.ShapeDtypeStruct((1, 1), jnp.float32),
    out_specs=pl.BlockSpec(memory_space=pltpu.SMEM),
)


def kernel(prob, target, reward):
    tgt_flat = target.astype(jnp.int32).reshape(-1)
    sel = _sc_gather(prob, tgt_flat)
    loss = _tc_reduce(sel.reshape(B, S), reward.reshape(B, 1))
    return loss[0, 0]


# speedup vs baseline: 23.0644x; 1.0255x over previous
import jax
import jax.numpy as jnp
from jax import lax
from jax.experimental import pallas as pl
from jax.experimental.pallas import tpu as pltpu
from jax.experimental.pallas import tpu_sc as plsc
import functools

NC = 2
NS = 16
L = 16
B = 64
S = 16
V = 100000
N = B * S
RPW = N // (NC * NS)
W8 = 8


def _sc_gather_kernel(prob_hbm, tgt_hbm, rew_hbm, sel_hbm,
                      tgt_v, rew_v, win_v, out_v, sem):
    w = lax.axis_index("c") * NS + lax.axis_index("s")
    base = w * RPW

    pltpu.sync_copy(tgt_hbm.at[pl.ds(base, RPW)], tgt_v)
    pltpu.sync_copy(rew_hbm, rew_v)

    iota = lax.iota(jnp.int32, L)
    negi = jnp.full((L,), -1, jnp.int32)

    tks = []
    copies = []
    for k in range(RPW):
        tv = tgt_v[pl.ds((k // L) * L, L)]
        t = jnp.max(jnp.where(iota == (k % L), tv, negi))
        tks.append(t)
        tcol = pl.multiple_of(jnp.bitwise_and(t, -128), 128)
        bb = 2 * w + k // S
        sg = (k % S) // 8 * 8
        copies.append(pltpu.async_copy(
            prob_hbm.at[bb, pl.ds(sg, 8), pl.ds(tcol, 128)],
            win_v.at[k], sem))
    for cp in copies:
        cp.wait()

    ninf = jnp.full((L,), -jnp.inf, jnp.float32)
    b0 = 2 * w
    rbase = pl.multiple_of(jnp.bitwise_and(b0, -L), L)
    rvec = rew_v[pl.ds(rbase, L)]
    rlane = jnp.bitwise_and(b0, L - 1)
    rews = [jnp.max(jnp.where(iota == rlane + h, rvec, ninf))
            for h in range(2)]

    for half in range(RPW // L):
        acc = jnp.zeros((L,), jnp.float32)
        for kk in range(L):
            k = half * L + kk
            lw = pl.multiple_of(
                jnp.bitwise_and(jnp.bitwise_and(tks[k], 127), -L), L)
            lanepos = jnp.bitwise_and(tks[k], L - 1)
            w16 = win_v.at[k, k % 8][pl.ds(lw, L)]
            v = jnp.max(jnp.where(iota == lanepos, w16, ninf))
            acc = jnp.where(iota == kk, jnp.full((L,), v), acc)
        out_v[pl.ds(half * L, L)] = acc * jnp.full((L,), rews[half])

    pltpu.sync_copy(out_v, sel_hbm.at[pl.ds(base, RPW)])


@functools.partial(
    pl.kernel,
    out_type=jax.ShapeDtypeStruct((N,), jnp.float32),
    mesh=plsc.VectorSubcoreMesh(
        core_axis_name="c", subcore_axis_name="s",
        num_cores=NC, num_subcores=NS),
    compiler_params=pltpu.CompilerParams(
        needs_layout_passes=False, use_tc_tiling_on_sc=True),
    scratch_types=[
        pltpu.VMEM((RPW,), jnp.int32),
        pltpu.VMEM((B,), jnp.float32),
        pltpu.VMEM((RPW, 8, 128), jnp.float32),
        pltpu.VMEM((RPW,), jnp.float32),
        pltpu.SemaphoreType.DMA,
    ],
)
def _sc_gather(prob_hbm, tgt_hbm, rew_hbm, sel_hbm, *scratch):
    _sc_gather_kernel(prob_hbm, tgt_hbm, rew_hbm, sel_hbm, *scratch)


def _tc_reduce_kernel(sel_ref, out_ref):
    out_ref[0, 0] = -jnp.sum(sel_ref[...])


_tc_reduce = pl.pallas_call(
    _tc_reduce_kernel,
    out_shape=jax.ShapeDtypeStruct((1, 1), jnp.float32),
    out_specs=pl.BlockSpec(memory_space=pltpu.SMEM),
)


def kernel(prob, target, reward):
    tgt_flat = target.astype(jnp.int32).reshape(-1)
    sel = _sc_gather(prob, tgt_flat, reward)
    loss = _tc_reduce(sel)
    return loss[0, 0]


# speedup vs baseline: 23.1162x; 1.0022x over previous
import jax
import jax.numpy as jnp
from jax import lax
from jax.experimental import pallas as pl
from jax.experimental.pallas import tpu as pltpu
from jax.experimental.pallas import tpu_sc as plsc
import functools

NC = 2
NS = 16
L = 16
B = 64
S = 16
V = 100000
N = B * S
RPW = N // (NC * NS)
W8 = 8


def _sc_gather_kernel(prob_hbm, tgt_hbm, rew_hbm, sel_hbm,
                      tgt_v, rew_v, win_v, out_v, sem):
    w = lax.axis_index("c") * NS + lax.axis_index("s")
    base = w * RPW

    pltpu.sync_copy(tgt_hbm.at[pl.ds(base, RPW)], tgt_v)
    pltpu.sync_copy(rew_hbm, rew_v)

    iota = lax.iota(jnp.int32, L)
    negi = jnp.full((L,), -1, jnp.int32)

    tks = []
    copies = []
    for k in range(RPW):
        tv = tgt_v[pl.ds((k // L) * L, L)]
        t = jnp.max(jnp.where(iota == (k % L), tv, negi))
        tks.append(t)
        tcol = pl.multiple_of(jnp.bitwise_and(t, -128), 128)
        bb = 2 * w + k // S
        sg = (k % S) // 8 * 8
        copies.append(pltpu.async_copy(
            prob_hbm.at[bb, pl.ds(sg, 8), pl.ds(tcol, 128)],
            win_v.at[k], sem))
    for cp in copies:
        cp.wait()

    ninf = jnp.full((L,), -jnp.inf, jnp.float32)
    b0 = 2 * w
    rbase = pl.multiple_of(jnp.bitwise_and(b0, -L), L)
    rvec = rew_v[pl.ds(rbase, L)]
    rlane = jnp.bitwise_and(b0, L - 1)
    rews = [jnp.max(jnp.where(iota == rlane + h, rvec, ninf))
            for h in range(2)]

    for half in range(RPW // L):
        acc = jnp.zeros((L,), jnp.float32)
        for kk in range(L):
            k = half * L + kk
            lw = pl.multiple_of(
                jnp.bitwise_and(jnp.bitwise_and(tks[k], 127), -L), L)
            lanepos = jnp.bitwise_and(tks[k], L - 1)
            w16 = win_v.at[k, k % 8][pl.ds(lw, L)]
            v = jnp.max(jnp.where(iota == lanepos, w16, ninf))
            acc = jnp.where(iota == kk, jnp.full((L,), v), acc)
        out_v[pl.ds(half * L, L)] = acc * jnp.full((L,), rews[half])

    pltpu.sync_copy(out_v, sel_hbm.at[pl.ds(base, RPW)])


@functools.partial(
    pl.kernel,
    out_type=jax.ShapeDtypeStruct((N,), jnp.float32),
    mesh=plsc.VectorSubcoreMesh(
        core_axis_name="c", subcore_axis_name="s",
        num_cores=NC, num_subcores=NS),
    compiler_params=pltpu.CompilerParams(
        needs_layout_passes=False, use_tc_tiling_on_sc=True,
        skip_device_barrier=True),
    scratch_types=[
        pltpu.VMEM((RPW,), jnp.int32),
        pltpu.VMEM((B,), jnp.float32),
        pltpu.VMEM((RPW, 8, 128), jnp.float32),
        pltpu.VMEM((RPW,), jnp.float32),
        pltpu.SemaphoreType.DMA,
    ],
)
def _sc_gather(prob_hbm, tgt_hbm, rew_hbm, sel_hbm, *scratch):
    _sc_gather_kernel(prob_hbm, tgt_hbm, rew_hbm, sel_hbm, *scratch)


def _tc_reduce_kernel(sel_ref, out_ref):
    out_ref[0, 0] = -jnp.sum(sel_ref[...])


_tc_reduce = pl.pallas_call(
    _tc_reduce_kernel,
    out_shape=jax.ShapeDtypeStruct((1, 1), jnp.float32),
    out_specs=pl.BlockSpec(memory_space=pltpu.SMEM),
)


def kernel(prob, target, reward):
    tgt_flat = target.astype(jnp.int32).reshape(-1)
    sel = _sc_gather(prob, tgt_flat, reward)
    loss = _tc_reduce(sel)
    return loss[0, 0]


# speedup vs baseline: 23.4000x; 1.0123x over previous
import jax
import jax.numpy as jnp
from jax import lax
from jax.experimental import pallas as pl
from jax.experimental.pallas import tpu as pltpu
from jax.experimental.pallas import tpu_sc as plsc
import functools

NC = 2
NS = 16
NW = NC * NS
L = 16
B = 64
S = 16
V = 100000
N = B * S
RPW = N // NW
WL = 128


def _sc_gather_kernel(prob_hbm, tgt_hbm, rew_hbm, sel_hbm,
                      tgt_v, rew_v, win_v, out_v, sem):
    w = lax.axis_index("c") * NS + lax.axis_index("s")
    base = pl.multiple_of(w * RPW, RPW)
    b0 = 2 * w

    pltpu.sync_copy(tgt_hbm, tgt_v)
    pltpu.sync_copy(rew_hbm, rew_v)

    iota = lax.iota(jnp.int32, L)
    negi = jnp.full((L,), -1, jnp.int32)

    tks = []
    copies = []
    for k in range(RPW):
        tv = tgt_v[pl.ds(base + (k // L) * L, L)]
        t = jnp.max(jnp.where(iota == (k % L), tv, negi))
        tks.append(t)
        tcol = pl.multiple_of(jnp.bitwise_and(t, -WL), WL)
        copies.append(pltpu.async_copy(
            prob_hbm.at[b0 + k // S, k % S, pl.ds(tcol, WL)],
            win_v.at[k], sem))
    for cp in copies:
        cp.wait()

    ninf = jnp.full((L,), -jnp.inf, jnp.float32)
    rbase = pl.multiple_of(jnp.bitwise_and(b0, -L), L)
    rvec = rew_v[pl.ds(rbase, L)]
    rlane = jnp.bitwise_and(b0, L - 1)
    rews = [jnp.max(jnp.where(iota == rlane + h, rvec, ninf))
            for h in range(2)]

    for half in range(RPW // L):
        acc = jnp.zeros((L,), jnp.float32)
        for kk in range(L):
            k = half * L + kk
            t = tks[k]
            lw = pl.multiple_of(jnp.bitwise_and(t, WL - L), L)
            w16 = win_v.at[k][pl.ds(lw, L)]
            lanepos = jnp.bitwise_and(t, L - 1)
            v = jnp.max(jnp.where(iota == lanepos, w16, ninf))
            acc = jnp.where(iota == kk, jnp.full((L,), v), acc)
        out_v[pl.ds(half * L, L)] = acc * jnp.full((L,), rews[half])
    for g in range(RPW // L, WL // L):
        out_v[pl.ds(g * L, L)] = jnp.zeros((L,), jnp.float32)

    pltpu.sync_copy(out_v, sel_hbm.at[w])


@functools.partial(
    pl.kernel,
    out_type=jax.ShapeDtypeStruct((NW, WL), jnp.float32),
    mesh=plsc.VectorSubcoreMesh(
        core_axis_name="c", subcore_axis_name="s",
        num_cores=NC, num_subcores=NS),
    compiler_params=pltpu.CompilerParams(
        needs_layout_passes=False, use_tc_tiling_on_sc=True,
        skip_device_barrier=True),
    scratch_types=[
        pltpu.VMEM((N,), jnp.int32),
        pltpu.VMEM((B,), jnp.float32),
        pltpu.VMEM((RPW, WL), jnp.float32),
        pltpu.VMEM((WL,), jnp.float32),
        pltpu.SemaphoreType.DMA,
    ],
)
def _sc_gather(prob_hbm, tgt_hbm, rew_hbm, sel_hbm, *scratch):
    _sc_gather_kernel(prob_hbm, tgt_hbm, rew_hbm, sel_hbm, *scratch)


def _tc_reduce_kernel(sel_ref, out_ref):
    out_ref[0, 0] = -jnp.sum(sel_ref[...])


_tc_reduce = pl.pallas_call(
    _tc_reduce_kernel,
    out_shape=jax.ShapeDtypeStruct((1, 1), jnp.float32),
    out_specs=pl.BlockSpec(memory_space=pltpu.SMEM),
)


def kernel(prob, target, reward):
    sel = _sc_gather(prob, target.astype(jnp.int32).reshape(-1), reward)
    loss = _tc_reduce(sel)
    return loss[0, 0]
